# baseline (speedup 1.0000x reference)
import functools

import jax
import jax.numpy as jnp
from jax import lax
from jax.experimental import pallas as pl
from jax.experimental.pallas import tpu as pltpu
from jax.experimental.pallas import tpu_sc as plsc

NUM_USERS = 100001
NUM_MEDS = 1001
EMBED_DIM = 32
BATCH = 4096
MAX_NORM = 1.0
EPS = 1e-7

_NC = 2
_NS = 16
_NW = _NC * _NS
_BPW = BATCH // _NW


def _sc_gather(user_table, user_idx, med_table, med_idx):
    mesh = plsc.VectorSubcoreMesh(core_axis_name="c", subcore_axis_name="s")

    @functools.partial(
        pl.kernel,
        out_type=[
            jax.ShapeDtypeStruct((BATCH, EMBED_DIM), jnp.float32),
            jax.ShapeDtypeStruct((BATCH, EMBED_DIM), jnp.float32),
        ],
        mesh=mesh,
        compiler_params=pltpu.CompilerParams(use_tc_tiling_on_sc=False),
        scratch_types=[
            pltpu.VMEM((_BPW,), jnp.int32),
            pltpu.VMEM((_BPW, EMBED_DIM), jnp.float32),
            pltpu.VMEM((_BPW,), jnp.int32),
            pltpu.VMEM((_BPW, EMBED_DIM), jnp.float32),
            pltpu.SemaphoreType.DMA,
            pltpu.SemaphoreType.DMA,
        ],
    )
    def gather_kernel(u_tab, u_idx, m_tab, m_idx, u_out, m_out,
                      uidx_v, urows_v, midx_v, mrows_v, usem, msem):
        wid = lax.axis_index("s") * _NC + lax.axis_index("c")
        base = wid * _BPW
        pltpu.sync_copy(u_idx.at[pl.ds(base, _BPW)], uidx_v)
        pltpu.sync_copy(m_idx.at[pl.ds(base, _BPW)], midx_v)
        cu = pltpu.async_copy(u_tab.at[uidx_v], urows_v, usem)
        cm = pltpu.async_copy(m_tab.at[midx_v], mrows_v, msem)
        cu.wait()
        cm.wait()
        pltpu.sync_copy(urows_v, u_out.at[pl.ds(base, _BPW)])
        pltpu.sync_copy(mrows_v, m_out.at[pl.ds(base, _BPW)])

    return gather_kernel(user_table, user_idx, med_table, med_idx)


def _renorm(x):
    norm = jnp.sqrt(jnp.sum(x * x, axis=-1, keepdims=True))
    scale = jnp.where(norm > MAX_NORM, MAX_NORM / (norm + EPS), 1.0)
    return x * scale


_M_TILE = 512


def _mm_body(u_ref, m_ref, o_ref):
    un = _renorm(u_ref[:])
    mn = _renorm(m_ref[:])
    o_ref[:] = lax.dot_general(
        un, mn, (((1,), (1,)), ((), ())), preferred_element_type=jnp.float32)


def _tc_matmul(u_emb, m_emb):
    grid = (BATCH // _M_TILE,)
    return pl.pallas_call(
        _mm_body,
        grid=grid,
        in_specs=[
            pl.BlockSpec((_M_TILE, EMBED_DIM), lambda i: (i, 0)),
            pl.BlockSpec((BATCH, EMBED_DIM), lambda i: (0, 0)),
        ],
        out_specs=pl.BlockSpec((_M_TILE, BATCH), lambda i: (i, 0)),
        out_shape=jax.ShapeDtypeStruct((BATCH, BATCH), jnp.float32),
    )(u_emb, m_emb)


def kernel(user_list, medicine_list, user_table, med_table):
    u_idx = user_list.astype(jnp.int32)
    m_idx = medicine_list.astype(jnp.int32)
    u_emb, m_emb = _sc_gather(user_table, u_idx, med_table, m_idx)
    return _tc_matmul(u_emb, m_emb)
